# baseline (speedup 1.0000x reference)
import jax
import jax.numpy as jnp
from jax import lax
from jax.experimental import pallas as pl
from jax.experimental.pallas import tpu as pltpu
from jax.experimental.pallas import tpu_sc as plsc

NC = 2
NS = 16
NW = NC * NS
CHUNK = 320
NCH = 32
EPW = CHUNK * NCH
NPAD = 10240
D = 128
BLK = 1280
RPT = NPAD // NS

_MESH = plsc.VectorSubcoreMesh(
    core_axis_name="c", subcore_axis_name="s", num_cores=NC, num_subcores=NS
)


def _deg_body(srcf, ones_hbm, zeros_hbm, deg_out, idxv, ones_v, acc):
    c = lax.axis_index("c")
    s = lax.axis_index("s")
    wid = c * NS + s
    pltpu.sync_copy(zeros_hbm.at[pl.ds(s * RPT, RPT)], acc.at[pl.ds(s * RPT, RPT)])
    pltpu.sync_copy(srcf.at[wid], idxv)
    pltpu.sync_copy(ones_hbm, ones_v)
    plsc.subcore_barrier()
    pltpu.sync_copy(ones_v, acc.at[idxv], add=True)
    plsc.subcore_barrier()
    pltpu.sync_copy(acc.at[pl.ds(s * RPT, RPT)], deg_out.at[c, pl.ds(s * RPT, RPT)])


def _agg_body(srcr, dstr, h_hbm, zeros_hbm, agg_out, src0, dst0, rows0, acc, gs0):
    c = lax.axis_index("c")
    s = lax.axis_index("s")
    wid = c * NS + s
    pltpu.sync_copy(zeros_hbm.at[pl.ds(s * RPT, RPT)], acc.at[pl.ds(s * RPT, RPT)])
    plsc.subcore_barrier()

    def step(g, carry):
        pltpu.sync_copy(srcr.at[wid, g], src0)
        pltpu.sync_copy(dstr.at[wid, g], dst0)
        pltpu.async_copy(h_hbm.at[src0], rows0, gs0).wait()
        pltpu.sync_copy(rows0, acc.at[dst0], add=True)
        return carry

    lax.fori_loop(0, NCH, step, 0)
    plsc.subcore_barrier()
    pltpu.sync_copy(acc.at[pl.ds(s * RPT, RPT)], agg_out.at[c, pl.ds(s * RPT, RPT)])


def _scale_body(deg_ref, emb_ref, h_ref):
    deg = deg_ref[0] + deg_ref[1]
    normed = lax.rsqrt(jnp.maximum(deg, 1.0))
    h_ref[...] = emb_ref[...] * normed[:, None]


def _final_body(deg_ref, agg_ref, out_ref):
    deg = deg_ref[0] + deg_ref[1]
    normed = lax.rsqrt(jnp.maximum(deg, 1.0))
    out_ref[...] = (agg_ref[0] + agg_ref[1]) * normed[:, None]


def kernel(users_emb, items_emb, edge_index):
    n_users, d = users_emb.shape
    n_items = items_emb.shape[0]
    n = n_users + n_items
    e = edge_index.shape[1]
    e_pad = NW * EPW
    assert d == D and n <= NPAD and e <= e_pad

    ei = edge_index.astype(jnp.int32)
    pad_idx = jnp.full((e_pad - e,), NPAD - 1, jnp.int32)
    src = jnp.concatenate([ei[0], pad_idx])
    dst = jnp.concatenate([ei[1], pad_idx])
    srcf = src.reshape(NW, EPW)
    srcr = src.reshape(NW, NCH, CHUNK)
    dstr = dst.reshape(NW, NCH, CHUNK)
    all_emb = jnp.concatenate(
        [users_emb, items_emb, jnp.zeros((NPAD - n, D), jnp.float32)], axis=0
    )
    ones = jnp.ones((EPW,), jnp.float32)
    zeros1 = jnp.zeros((NPAD,), jnp.float32)
    zeros128 = jnp.zeros((NPAD, D), jnp.float32)

    deg_parts = pl.kernel(
        _deg_body,
        out_type=jax.ShapeDtypeStruct((NC, NPAD), jnp.float32),
        mesh=_MESH,
        scratch_types=[
            pltpu.VMEM((EPW,), jnp.int32),
            pltpu.VMEM((EPW,), jnp.float32),
            pltpu.VMEM_SHARED((NPAD,), jnp.float32),
        ],
    )(srcf, ones, zeros1)

    h = pl.pallas_call(
        _scale_body,
        grid=(NPAD // BLK,),
        in_specs=[
            pl.BlockSpec((2, BLK), lambda i: (0, i)),
            pl.BlockSpec((BLK, D), lambda i: (i, 0)),
        ],
        out_specs=pl.BlockSpec((BLK, D), lambda i: (i, 0)),
        out_shape=jax.ShapeDtypeStruct((NPAD, D), jnp.float32),
    )(deg_parts, all_emb)

    agg_parts = pl.kernel(
        _agg_body,
        out_type=jax.ShapeDtypeStruct((NC, NPAD, D), jnp.float32),
        mesh=_MESH,
        scratch_types=[
            pltpu.VMEM((CHUNK,), jnp.int32),
            pltpu.VMEM((CHUNK,), jnp.int32),
            pltpu.VMEM((CHUNK, D), jnp.float32),
            pltpu.VMEM_SHARED((NPAD, D), jnp.float32),
            pltpu.SemaphoreType.DMA,
        ],
    )(srcr, dstr, h, zeros128)

    out = pl.pallas_call(
        _final_body,
        grid=(NPAD // BLK,),
        in_specs=[
            pl.BlockSpec((2, BLK), lambda i: (0, i)),
            pl.BlockSpec((2, BLK, D), lambda i: (0, i, 0)),
        ],
        out_specs=pl.BlockSpec((BLK, D), lambda i: (i, 0)),
        out_shape=jax.ShapeDtypeStruct((NPAD, D), jnp.float32),
    )(deg_parts, agg_parts)

    return out[:n]


# speedup vs baseline: 2.2643x; 2.2643x over previous
import jax
import jax.numpy as jnp
from jax import lax
from jax.experimental import pallas as pl
from jax.experimental.pallas import tpu as pltpu
from jax.experimental.pallas import tpu_sc as plsc

NC = 2
NS = 16
NW = NC * NS
CHUNK = 320
NCH = 32
EPW = CHUNK * NCH
NPAD = 10240
D = 128
BLK = 1280
RPT = NPAD // NS

_MESH = plsc.VectorSubcoreMesh(
    core_axis_name="c", subcore_axis_name="s", num_cores=NC, num_subcores=NS
)


def _deg_body(srcf, ones_hbm, zeros_hbm, deg_out, idxv, ones_v, acc):
    c = lax.axis_index("c")
    s = lax.axis_index("s")
    wid = c * NS + s
    pltpu.sync_copy(zeros_hbm.at[pl.ds(s * RPT, RPT)], acc.at[pl.ds(s * RPT, RPT)])
    pltpu.sync_copy(srcf.at[wid], idxv)
    pltpu.sync_copy(ones_hbm, ones_v)
    plsc.subcore_barrier()
    pltpu.sync_copy(ones_v, acc.at[idxv], add=True)
    plsc.subcore_barrier()
    pltpu.sync_copy(acc.at[pl.ds(s * RPT, RPT)], deg_out.at[c, pl.ds(s * RPT, RPT)])


def _agg_body(srcr, dstr, h_hbm, zeros_hbm, agg_out, src0, dst0, rows0, acc, gs0):
    c = lax.axis_index("c")
    s = lax.axis_index("s")
    wid = c * NS + s
    pltpu.sync_copy(zeros_hbm.at[pl.ds(s * RPT, RPT)], acc.at[pl.ds(s * RPT, RPT)])
    plsc.subcore_barrier()

    def step(g, carry):
        pltpu.sync_copy(srcr.at[wid, g], src0)
        pltpu.sync_copy(dstr.at[wid, g], dst0)
        pltpu.async_copy(h_hbm.at[src0], rows0, gs0).wait()
        pltpu.sync_copy(rows0, acc.at[dst0], add=True)
        return carry

    lax.fori_loop(0, NCH, step, 0)
    plsc.subcore_barrier()
    pltpu.sync_copy(acc.at[pl.ds(s * RPT, RPT)], agg_out.at[c, pl.ds(s * RPT, RPT)])


def _scale_body(deg_ref, emb_ref, h_ref):
    deg = deg_ref[0] + deg_ref[1]
    normed = lax.rsqrt(jnp.maximum(deg, 1.0))
    h_ref[...] = emb_ref[...] * normed[:, None]


def _final_body(deg_ref, agg_ref, out_ref):
    deg = deg_ref[0] + deg_ref[1]
    normed = lax.rsqrt(jnp.maximum(deg, 1.0))
    out_ref[...] = (agg_ref[0] + agg_ref[1]) * normed[:, None]


def kernel(users_emb, items_emb, edge_index):
    n_users, d = users_emb.shape
    n_items = items_emb.shape[0]
    n = n_users + n_items
    e = edge_index.shape[1]
    e_pad = NW * EPW
    assert d == D and n <= NPAD and e <= e_pad

    ei = edge_index.astype(jnp.int32)
    pad_idx = n + jnp.arange(e_pad - e, dtype=jnp.int32) % (NPAD - n)
    src = jnp.concatenate([ei[0], pad_idx])
    dst = jnp.concatenate([ei[1], pad_idx])
    srcf = src.reshape(NW, EPW)
    srcr = src.reshape(NW, NCH, CHUNK)
    dstr = dst.reshape(NW, NCH, CHUNK)
    all_emb = jnp.concatenate(
        [users_emb, items_emb, jnp.zeros((NPAD - n, D), jnp.float32)], axis=0
    )
    ones = jnp.ones((EPW,), jnp.float32)
    zeros1 = jnp.zeros((NPAD,), jnp.float32)
    zeros128 = jnp.zeros((NPAD, D), jnp.float32)

    deg_parts = pl.kernel(
        _deg_body,
        out_type=jax.ShapeDtypeStruct((NC, NPAD), jnp.float32),
        mesh=_MESH,
        scratch_types=[
            pltpu.VMEM((EPW,), jnp.int32),
            pltpu.VMEM((EPW,), jnp.float32),
            pltpu.VMEM_SHARED((NPAD,), jnp.float32),
        ],
    )(srcf, ones, zeros1)

    h = pl.pallas_call(
        _scale_body,
        grid=(NPAD // BLK,),
        in_specs=[
            pl.BlockSpec((2, BLK), lambda i: (0, i)),
            pl.BlockSpec((BLK, D), lambda i: (i, 0)),
        ],
        out_specs=pl.BlockSpec((BLK, D), lambda i: (i, 0)),
        out_shape=jax.ShapeDtypeStruct((NPAD, D), jnp.float32),
    )(deg_parts, all_emb)

    agg_parts = pl.kernel(
        _agg_body,
        out_type=jax.ShapeDtypeStruct((NC, NPAD, D), jnp.float32),
        mesh=_MESH,
        scratch_types=[
            pltpu.VMEM((CHUNK,), jnp.int32),
            pltpu.VMEM((CHUNK,), jnp.int32),
            pltpu.VMEM((CHUNK, D), jnp.float32),
            pltpu.VMEM_SHARED((NPAD, D), jnp.float32),
            pltpu.SemaphoreType.DMA,
        ],
    )(srcr, dstr, h, zeros128)

    out = pl.pallas_call(
        _final_body,
        grid=(NPAD // BLK,),
        in_specs=[
            pl.BlockSpec((2, BLK), lambda i: (0, i)),
            pl.BlockSpec((2, BLK, D), lambda i: (0, i, 0)),
        ],
        out_specs=pl.BlockSpec((BLK, D), lambda i: (i, 0)),
        out_shape=jax.ShapeDtypeStruct((NPAD, D), jnp.float32),
    )(deg_parts, agg_parts)

    return out[:n]


# speedup vs baseline: 2.4061x; 1.0626x over previous
import jax
import jax.numpy as jnp
from jax import lax
from jax.experimental import pallas as pl
from jax.experimental.pallas import tpu as pltpu
from jax.experimental.pallas import tpu_sc as plsc

NC = 2
NS = 16
NW = NC * NS
CHUNK = 160
NCH = 64
EPW = CHUNK * NCH
NPAD = 10240
D = 128
BLK = 1280
RPT = NPAD // NS

_MESH = plsc.VectorSubcoreMesh(
    core_axis_name="c", subcore_axis_name="s", num_cores=NC, num_subcores=NS
)


def _deg_body(srcf, ones_hbm, zeros_hbm, deg_out, idxv, ones_v, acc):
    c = lax.axis_index("c")
    s = lax.axis_index("s")
    wid = c * NS + s
    pltpu.sync_copy(zeros_hbm.at[pl.ds(s * RPT, RPT)], acc.at[pl.ds(s * RPT, RPT)])
    pltpu.sync_copy(srcf.at[wid], idxv)
    pltpu.sync_copy(ones_hbm, ones_v)
    plsc.subcore_barrier()
    pltpu.sync_copy(ones_v, acc.at[idxv], add=True)
    plsc.subcore_barrier()
    pltpu.sync_copy(acc.at[pl.ds(s * RPT, RPT)], deg_out.at[c, pl.ds(s * RPT, RPT)])


def _agg_body(srcr, dstr, h_hbm, zeros_hbm, agg_out,
              src0, dst0, src1, dst1, rows0, rows1, acc, gs0, gs1):
    c = lax.axis_index("c")
    s = lax.axis_index("s")
    wid = c * NS + s
    pltpu.sync_copy(zeros_hbm.at[pl.ds(s * RPT, RPT)], acc.at[pl.ds(s * RPT, RPT)])
    plsc.subcore_barrier()

    def step(i, carry):
        g0 = 2 * i
        g1 = 2 * i + 1
        pltpu.sync_copy(srcr.at[wid, g0], src0)
        pltpu.sync_copy(dstr.at[wid, g0], dst0)
        a0 = pltpu.async_copy(h_hbm.at[src0], rows0, gs0)
        pltpu.sync_copy(srcr.at[wid, g1], src1)
        pltpu.sync_copy(dstr.at[wid, g1], dst1)
        a1 = pltpu.async_copy(h_hbm.at[src1], rows1, gs1)
        a0.wait()
        pltpu.sync_copy(rows0, acc.at[dst0], add=True)
        a1.wait()
        pltpu.sync_copy(rows1, acc.at[dst1], add=True)
        return carry

    lax.fori_loop(0, NCH // 2, step, 0)
    plsc.subcore_barrier()
    pltpu.sync_copy(acc.at[pl.ds(s * RPT, RPT)], agg_out.at[c, pl.ds(s * RPT, RPT)])


def _scale_body(deg_ref, emb_ref, h_ref):
    deg = deg_ref[0] + deg_ref[1]
    normed = lax.rsqrt(jnp.maximum(deg, 1.0))
    h_ref[...] = emb_ref[...] * normed[:, None]


def _final_body(deg_ref, agg_ref, out_ref):
    deg = deg_ref[0] + deg_ref[1]
    normed = lax.rsqrt(jnp.maximum(deg, 1.0))
    out_ref[...] = (agg_ref[0] + agg_ref[1]) * normed[:, None]


def kernel(users_emb, items_emb, edge_index):
    n_users, d = users_emb.shape
    n_items = items_emb.shape[0]
    n = n_users + n_items
    e = edge_index.shape[1]
    e_pad = NW * EPW
    assert d == D and n <= NPAD and e <= e_pad

    ei = edge_index.astype(jnp.int32)
    pad_idx = n + jnp.arange(e_pad - e, dtype=jnp.int32) % (NPAD - n)
    src = jnp.concatenate([ei[0], pad_idx])
    dst = jnp.concatenate([ei[1], pad_idx])
    srcf = src.reshape(NW, EPW)
    srcr = src.reshape(NW, NCH, CHUNK)
    dstr = dst.reshape(NW, NCH, CHUNK)
    all_emb = jnp.concatenate(
        [users_emb, items_emb, jnp.zeros((NPAD - n, D), jnp.float32)], axis=0
    )
    ones = jnp.ones((EPW,), jnp.float32)
    zeros1 = jnp.zeros((NPAD,), jnp.float32)
    zeros128 = jnp.zeros((NPAD, D), jnp.float32)

    deg_parts = pl.kernel(
        _deg_body,
        out_type=jax.ShapeDtypeStruct((NC, NPAD), jnp.float32),
        mesh=_MESH,
        scratch_types=[
            pltpu.VMEM((EPW,), jnp.int32),
            pltpu.VMEM((EPW,), jnp.float32),
            pltpu.VMEM_SHARED((NPAD,), jnp.float32),
        ],
    )(srcf, ones, zeros1)

    h = pl.pallas_call(
        _scale_body,
        grid=(NPAD // BLK,),
        in_specs=[
            pl.BlockSpec((2, BLK), lambda i: (0, i)),
            pl.BlockSpec((BLK, D), lambda i: (i, 0)),
        ],
        out_specs=pl.BlockSpec((BLK, D), lambda i: (i, 0)),
        out_shape=jax.ShapeDtypeStruct((NPAD, D), jnp.float32),
    )(deg_parts, all_emb)

    agg_parts = pl.kernel(
        _agg_body,
        out_type=jax.ShapeDtypeStruct((NC, NPAD, D), jnp.float32),
        mesh=_MESH,
        scratch_types=[
            pltpu.VMEM((CHUNK,), jnp.int32),
            pltpu.VMEM((CHUNK,), jnp.int32),
            pltpu.VMEM((CHUNK,), jnp.int32),
            pltpu.VMEM((CHUNK,), jnp.int32),
            pltpu.VMEM((CHUNK, D), jnp.float32),
            pltpu.VMEM((CHUNK, D), jnp.float32),
            pltpu.VMEM_SHARED((NPAD, D), jnp.float32),
            pltpu.SemaphoreType.DMA,
            pltpu.SemaphoreType.DMA,
        ],
    )(srcr, dstr, h, zeros128)

    out = pl.pallas_call(
        _final_body,
        grid=(NPAD // BLK,),
        in_specs=[
            pl.BlockSpec((2, BLK), lambda i: (0, i)),
            pl.BlockSpec((2, BLK, D), lambda i: (0, i, 0)),
        ],
        out_specs=pl.BlockSpec((BLK, D), lambda i: (i, 0)),
        out_shape=jax.ShapeDtypeStruct((NPAD, D), jnp.float32),
    )(deg_parts, agg_parts)

    return out[:n]
